# baseline (speedup 1.0000x reference)
import functools

import jax
import jax.numpy as jnp
from jax import lax
from jax.experimental import pallas as pl
from jax.experimental.pallas import tpu as pltpu
from jax.experimental.pallas import tpu_sc as plsc

D = 128
BS = 512
K = 64
NC, NS = 2, 16
NW = NC * NS


def _pre_body(feat_ref, watt_ref, wattb_ref, a1_ref, a2_ref, w1_ref, w1b_ref,
              hmsg_ref, s_ref):
    f = feat_ref[...]
    ht = lax.dot_general(f, watt_ref[...], (((1,), (1,)), ((), ())),
                         preferred_element_type=jnp.float32) + wattb_ref[...]
    s1 = lax.dot_general(a1_ref[...], ht, (((1,), (1,)), ((), ())),
                         preferred_element_type=jnp.float32)
    s2 = lax.dot_general(a2_ref[...], ht, (((1,), (1,)), ((), ())),
                         preferred_element_type=jnp.float32)
    s_ref[0:1, :] = s1
    s_ref[1:2, :] = s2
    s_ref[2:8, :] = jnp.zeros((6, s1.shape[1]), jnp.float32)
    hmsg_ref[...] = lax.dot_general(f, w1_ref[...], (((1,), (1,)), ((), ())),
                                    preferred_element_type=jnp.float32) + w1b_ref[...]


def _tc_pre(featp, Watt_w, Watt_b, a1, a2, W1_w, W1_b):
    NP = featp.shape[0]
    grid = (NP // BS,)
    return pl.pallas_call(
        _pre_body,
        grid=grid,
        in_specs=[
            pl.BlockSpec((BS, D), lambda i: (i, 0)),
            pl.BlockSpec((D, D), lambda i: (0, 0)),
            pl.BlockSpec((1, D), lambda i: (0, 0)),
            pl.BlockSpec((1, D), lambda i: (0, 0)),
            pl.BlockSpec((1, D), lambda i: (0, 0)),
            pl.BlockSpec((D, D), lambda i: (0, 0)),
            pl.BlockSpec((1, D), lambda i: (0, 0)),
        ],
        out_specs=[
            pl.BlockSpec((BS, D), lambda i: (i, 0)),
            pl.BlockSpec((8, BS), lambda i: (0, i)),
        ],
        out_shape=[
            jax.ShapeDtypeStruct((NP, D), jnp.float32),
            jax.ShapeDtypeStruct((8, NP), jnp.float32),
        ],
    )(featp, Watt_w, Watt_b, a1, a2, W1_w, W1_b)


SUP = 384
NCK = SUP // K


def _sc_edge_call(src, dst, s_out, hmsg, NP, N, E, EPW0, EPW1):
    npairs0, npairs1 = EPW0 // (2 * SUP), EPW1 // (2 * SUP)
    stripe = NP // NS
    mesh = plsc.VectorSubcoreMesh(core_axis_name="c", subcore_axis_name="s")

    def body(src_hbm, dst_hbm, s_hbm, hmsg_hbm, acc_out, den_out,
             s1_v, s2_v, den_v, ssA, dsA, ssB, dsB, db0, db1, w0, w1,
             rows0, rows1, acc_sh, gsem0, gsem1, ldA, ldB):
        dbs, wvs, rows, gsems = [db0, db1], [w0, w1], [rows0, rows1], [gsem0, gsem1]
        c = lax.axis_index("c")
        s = lax.axis_index("s")
        wid = s * NC + c
        ebase = jnp.where(c == 0, s * EPW0, NS * EPW0 + s * EPW1)
        npairs_c = jnp.where(c == 0, npairs0, npairs1)

        pltpu.sync_copy(s_hbm.at[0], s1_v)
        pltpu.sync_copy(s_hbm.at[1], s2_v)

        def _zden(i, _):
            den_v[pl.ds(i * 16, 16)] = jnp.zeros((16,), jnp.float32)
            return _
        lax.fori_loop(0, N // 16, _zden, 0)

        def _zrow(i, _):
            for j in range(D // 16):
                rows0[i, pl.ds(j * 16, 16)] = jnp.zeros((16,), jnp.float32)
            return _
        lax.fori_loop(0, K, _zrow, 0)
        for t in range(stripe // K):
            pltpu.sync_copy(rows0, acc_sh.at[pl.ds(s * stripe + t * K, K)])
        plsc.subcore_barrier()

        iota16 = lax.broadcasted_iota(jnp.int32, (16,), 0)

        def scalar_phase(gbase, ss_c, ds_c, ck, par):
            for j in range(K // 16):
                off = ck * K + j * 16
                si = ss_c[pl.ds(off, 16)]
                di = ds_c[pl.ds(off, 16)]
                z = plsc.load_gather(s1_v, [si]) + plsc.load_gather(s2_v, [di])
                z = jnp.where(z >= 0.0, z, 0.2 * z)
                w = jnp.exp(z)
                gid = gbase + off + iota16
                w = jnp.where(gid < E, w, 0.0)
                wvs[par][pl.ds(j * 16, 16)] = w
                dbs[par][pl.ds(j * 16, 16)] = di
                plsc.addupdate_scatter(den_v, [di], w)

        def scale_rows(par):
            rws, wv = rows[par], wvs[par]

            def scale(k, _s):
                for u in range(4):
                    kk = k * 4 + u
                    wk = plsc.load_gather(wv, [lax.broadcast(kk, (16,))])
                    for j in range(D // 16):
                        rws[kk, pl.ds(j * 16, 16)] = rws[kk, pl.ds(j * 16, 16)] * wk
                return _s
            lax.fori_loop(0, K // 4, scale, 0)

        def do_super(sbase, ss_c, ds_c, ss_n, ds_n, ld_n, fire_guard):
            @pl.when(fire_guard)
            def _prefetch():
                nb = sbase + SUP
                pltpu.async_copy(src_hbm.at[pl.ds(nb, SUP)], ss_n, ld_n)
                pltpu.async_copy(dst_hbm.at[pl.ds(nb, SUP)], ds_n, ld_n)

            for ck in range(NCK):
                par = ck % 2
                if ck < NCK - 1:
                    pltpu.async_copy(
                        hmsg_hbm.at[ss_c.at[pl.ds((ck + 1) * K, K)]],
                        rows[1 - par], gsems[1 - par])
                else:
                    @pl.when(fire_guard)
                    def _boundary():
                        pltpu.make_async_copy(src_hbm.at[pl.ds(0, SUP)],
                                              ss_n, ld_n).wait()
                        pltpu.make_async_copy(dst_hbm.at[pl.ds(0, SUP)],
                                              ds_n, ld_n).wait()
                        pltpu.async_copy(hmsg_hbm.at[ss_n.at[pl.ds(0, K)]],
                                         rows[1 - par], gsems[1 - par])
                scalar_phase(sbase, ss_c, ds_c, ck, par)
                pltpu.make_async_copy(hmsg_hbm.at[ss_c.at[pl.ds(ck * K, K)]],
                                      rows[par], gsems[par]).wait()
                scale_rows(par)
                pltpu.sync_copy(rows[par], acc_sh.at[dbs[par]], add=True)

        pltpu.sync_copy(src_hbm.at[pl.ds(ebase, SUP)], ssA)
        pltpu.sync_copy(dst_hbm.at[pl.ds(ebase, SUP)], dsA)
        pltpu.async_copy(hmsg_hbm.at[ssA.at[pl.ds(0, K)]], rows0, gsem0)

        def pair_body(i, _):
            base_a = ebase + (2 * i) * SUP
            do_super(base_a, ssA, dsA, ssB, dsB, ldB, i >= 0)
            do_super(base_a + SUP, ssB, dsB, ssA, dsA, ldA, i + 1 < npairs_c)
            return _
        lax.fori_loop(0, npairs_c, pair_body, 0)

        plsc.subcore_barrier()
        pltpu.sync_copy(acc_sh.at[pl.ds(s * stripe, stripe)],
                        acc_out.at[c, pl.ds(s * stripe, stripe)])
        pltpu.sync_copy(den_v, den_out.at[wid])

    fn = pl.kernel(
        body,
        out_type=[
            jax.ShapeDtypeStruct((NC, NP, D), jnp.float32),
            jax.ShapeDtypeStruct((NW, N), jnp.float32),
        ],
        mesh=mesh,
        compiler_params=pltpu.CompilerParams(needs_layout_passes=False),
        scratch_types=[
            pltpu.VMEM((NP,), jnp.float32),
            pltpu.VMEM((NP,), jnp.float32),
            pltpu.VMEM((N,), jnp.float32),
            pltpu.VMEM((SUP,), jnp.int32),
            pltpu.VMEM((SUP,), jnp.int32),
            pltpu.VMEM((SUP,), jnp.int32),
            pltpu.VMEM((SUP,), jnp.int32),
            pltpu.VMEM((K,), jnp.int32),
            pltpu.VMEM((K,), jnp.int32),
            pltpu.VMEM((K,), jnp.float32),
            pltpu.VMEM((K,), jnp.float32),
            pltpu.VMEM((K, D), jnp.float32),
            pltpu.VMEM((K, D), jnp.float32),
            pltpu.VMEM_SHARED((NP, D), jnp.float32),
            pltpu.SemaphoreType.DMA,
            pltpu.SemaphoreType.DMA,
            pltpu.SemaphoreType.DMA,
            pltpu.SemaphoreType.DMA,
        ],
    )
    return fn(src, dst, s_out, hmsg)


def _post_body(acc_ref, den_ref, feat_ref, w2_ref, w2b_ref, out_ref):
    acc = acc_ref[0] + acc_ref[1]
    den = jnp.sum(den_ref[...], axis=0)[:, None]
    h = acc / (den + 1e-9)
    f = feat_ref[...]
    w2p = lax.dot_general(f * h, w2_ref[...], (((1,), (1,)), ((), ())),
                          preferred_element_type=jnp.float32) + w2b_ref[...]
    o = f + h + w2p
    out_ref[...] = jnp.where(o >= 0.0, o, 0.2 * o)


def _tc_post(acc, den, featp, W2_w, W2_b, N):
    NP = featp.shape[0]
    grid = (NP // BS,)
    return pl.pallas_call(
        _post_body,
        grid=grid,
        in_specs=[
            pl.BlockSpec((NC, BS, D), lambda i: (0, i, 0)),
            pl.BlockSpec((NW, BS), lambda i: (0, i)),
            pl.BlockSpec((BS, D), lambda i: (i, 0)),
            pl.BlockSpec((D, D), lambda i: (0, 0)),
            pl.BlockSpec((1, D), lambda i: (0, 0)),
        ],
        out_specs=pl.BlockSpec((BS, D), lambda i: (i, 0)),
        out_shape=jax.ShapeDtypeStruct((N, D), jnp.float32),
    )(acc, den, featp, W2_w, W2_b)


def kernel(indices, features, num_nodes, W1_w, W1_b, W2_w, W2_b, Watt_w, Watt_b, a):
    N = features.shape[0]
    E = indices.shape[1]
    NP = -(-N // BS) * BS
    per_s = -(-E // NS)
    g = 2 * SUP
    EPW0 = max(g, int(round(0.5 * per_s / g)) * g)
    EPW1 = -(-(per_s - EPW0) // g) * g
    EP = NS * (EPW0 + EPW1)

    idxp = jnp.pad(indices.astype(jnp.int32), ((0, 0), (0, EP - E)))
    featp = jnp.pad(features.astype(jnp.float32), ((0, NP - N), (0, 0)))
    a1 = a[:D, 0].reshape(1, D).astype(jnp.float32)
    a2 = a[D:, 0].reshape(1, D).astype(jnp.float32)

    hmsg, s_out = _tc_pre(featp, Watt_w, Watt_b.reshape(1, D), a1, a2,
                          W1_w, W1_b.reshape(1, D))
    acc, den = _sc_edge_call(idxp[0], idxp[1], s_out, hmsg, NP, N, E, EPW0, EPW1)
    out = _tc_post(acc, den, featp, W2_w, W2_b.reshape(1, D), N)
    return out


# speedup vs baseline: 1.9233x; 1.9233x over previous
import functools

import jax
import jax.numpy as jnp
from jax import lax
from jax.experimental import pallas as pl
from jax.experimental.pallas import tpu as pltpu
from jax.experimental.pallas import tpu_sc as plsc

D = 128
BS = 512
K = 64
NC, NS = 2, 16
NW = NC * NS


def _pre_body(feat_ref, watt_ref, wattb_ref, a1_ref, a2_ref, w1_ref, w1b_ref,
              hmsg_ref, s_ref):
    f = feat_ref[...]
    ht = lax.dot_general(f, watt_ref[...], (((1,), (1,)), ((), ())),
                         preferred_element_type=jnp.float32) + wattb_ref[...]
    s1 = lax.dot_general(a1_ref[...], ht, (((1,), (1,)), ((), ())),
                         preferred_element_type=jnp.float32)
    s2 = lax.dot_general(a2_ref[...], ht, (((1,), (1,)), ((), ())),
                         preferred_element_type=jnp.float32)
    s_ref[0:1, :] = s1
    s_ref[1:2, :] = s2
    s_ref[2:8, :] = jnp.zeros((6, s1.shape[1]), jnp.float32)
    hmsg_ref[...] = lax.dot_general(f, w1_ref[...], (((1,), (1,)), ((), ())),
                                    preferred_element_type=jnp.float32) + w1b_ref[...]


def _tc_pre(featp, Watt_w, Watt_b, a1, a2, W1_w, W1_b):
    NP = featp.shape[0]
    grid = (NP // BS,)
    return pl.pallas_call(
        _pre_body,
        grid=grid,
        in_specs=[
            pl.BlockSpec((BS, D), lambda i: (i, 0)),
            pl.BlockSpec((D, D), lambda i: (0, 0)),
            pl.BlockSpec((1, D), lambda i: (0, 0)),
            pl.BlockSpec((1, D), lambda i: (0, 0)),
            pl.BlockSpec((1, D), lambda i: (0, 0)),
            pl.BlockSpec((D, D), lambda i: (0, 0)),
            pl.BlockSpec((1, D), lambda i: (0, 0)),
        ],
        out_specs=[
            pl.BlockSpec((BS, D), lambda i: (i, 0)),
            pl.BlockSpec((8, BS), lambda i: (0, i)),
        ],
        out_shape=[
            jax.ShapeDtypeStruct((NP, D), jnp.float32),
            jax.ShapeDtypeStruct((8, NP), jnp.float32),
        ],
    )(featp, Watt_w, Watt_b, a1, a2, W1_w, W1_b)


SUP = 128
NCK = SUP // K


def _sc_edge_call(src, dst, s_out, hmsg, NP, N, E, EPW0, EPW1):
    npairs0, npairs1 = EPW0 // (2 * SUP), EPW1 // (2 * SUP)
    stripe = NP // NS
    mesh = plsc.VectorSubcoreMesh(core_axis_name="c", subcore_axis_name="s")

    def body(src_hbm, dst_hbm, s_hbm, hmsg_hbm, acc_out, den_out,
             s1_v, s2_v, den_v, ssA, dsA, ssB, dsB, db0, db1, w0, w1,
             rows0, rows1, acc_sh, gsem0, gsem1, ldA, ldB):
        dbs, wvs, rows, gsems = [db0, db1], [w0, w1], [rows0, rows1], [gsem0, gsem1]
        c = lax.axis_index("c")
        s = lax.axis_index("s")
        wid = s * NC + c
        ebase = jnp.where(c == 0, s * EPW0, NS * EPW0 + s * EPW1)
        npairs_c = jnp.where(c == 0, npairs0, npairs1)

        pltpu.sync_copy(s_hbm.at[0], s1_v)
        pltpu.sync_copy(s_hbm.at[1], s2_v)

        def _zden(i, _):
            den_v[pl.ds(i * 16, 16)] = jnp.zeros((16,), jnp.float32)
            return _
        lax.fori_loop(0, N // 16, _zden, 0)

        def _zrow(i, _):
            for j in range(D // 16):
                rows0[i, pl.ds(j * 16, 16)] = jnp.zeros((16,), jnp.float32)
            return _
        lax.fori_loop(0, K, _zrow, 0)
        for t in range(stripe // K):
            pltpu.sync_copy(rows0, acc_sh.at[pl.ds(s * stripe + t * K, K)])
        plsc.subcore_barrier()

        iota16 = lax.broadcasted_iota(jnp.int32, (16,), 0)

        def scalar_phase(gbase, ss_c, ds_c, ck, par):
            for j in range(K // 16):
                off = ck * K + j * 16
                si = ss_c[pl.ds(off, 16)]
                di = ds_c[pl.ds(off, 16)]
                z = plsc.load_gather(s1_v, [si]) + plsc.load_gather(s2_v, [di])
                z = jnp.where(z >= 0.0, z, 0.2 * z)
                w = jnp.exp(z)
                gid = gbase + off + iota16
                w = jnp.where(gid < E, w, 0.0)
                wvs[par][pl.ds(j * 16, 16)] = w
                dbs[par][pl.ds(j * 16, 16)] = di
                plsc.addupdate_scatter(den_v, [di], w)

        def scale_rows(par):
            rws, wv = rows[par], wvs[par]

            def scale(k, _s):
                for u in range(4):
                    kk = k * 4 + u
                    wk = plsc.load_gather(wv, [lax.broadcast(kk, (16,))])
                    for j in range(D // 16):
                        rws[kk, pl.ds(j * 16, 16)] = rws[kk, pl.ds(j * 16, 16)] * wk
                return _s
            lax.fori_loop(0, K // 4, scale, 0)

        def do_super(sbase, ss_c, ds_c, ss_n, ds_n, ld_n, fire_guard):
            @pl.when(fire_guard)
            def _prefetch():
                nb = sbase + SUP
                pltpu.async_copy(src_hbm.at[pl.ds(nb, SUP)], ss_n, ld_n)
                pltpu.async_copy(dst_hbm.at[pl.ds(nb, SUP)], ds_n, ld_n)

            for ck in range(NCK):
                par = ck % 2
                if ck < NCK - 1:
                    pltpu.async_copy(
                        hmsg_hbm.at[ss_c.at[pl.ds((ck + 1) * K, K)]],
                        rows[1 - par], gsems[1 - par])
                else:
                    @pl.when(fire_guard)
                    def _boundary():
                        pltpu.make_async_copy(src_hbm.at[pl.ds(0, SUP)],
                                              ss_n, ld_n).wait()
                        pltpu.make_async_copy(dst_hbm.at[pl.ds(0, SUP)],
                                              ds_n, ld_n).wait()
                        pltpu.async_copy(hmsg_hbm.at[ss_n.at[pl.ds(0, K)]],
                                         rows[1 - par], gsems[1 - par])
                scalar_phase(sbase, ss_c, ds_c, ck, par)
                pltpu.make_async_copy(hmsg_hbm.at[ss_c.at[pl.ds(ck * K, K)]],
                                      rows[par], gsems[par]).wait()
                scale_rows(par)
                pltpu.sync_copy(rows[par], acc_sh.at[dbs[par]], add=True)

        pltpu.sync_copy(src_hbm.at[pl.ds(ebase, SUP)], ssA)
        pltpu.sync_copy(dst_hbm.at[pl.ds(ebase, SUP)], dsA)
        pltpu.async_copy(hmsg_hbm.at[ssA.at[pl.ds(0, K)]], rows0, gsem0)

        def pair_body(i, _):
            base_a = ebase + (2 * i) * SUP
            do_super(base_a, ssA, dsA, ssB, dsB, ldB, i >= 0)
            do_super(base_a + SUP, ssB, dsB, ssA, dsA, ldA, i + 1 < npairs_c)
            return _
        lax.fori_loop(0, npairs_c, pair_body, 0)

        plsc.subcore_barrier()
        pltpu.sync_copy(acc_sh.at[pl.ds(s * stripe, stripe)],
                        acc_out.at[c, pl.ds(s * stripe, stripe)])
        pltpu.sync_copy(den_v, den_out.at[wid])

    fn = pl.kernel(
        body,
        out_type=[
            jax.ShapeDtypeStruct((NC, NP, D), jnp.float32),
            jax.ShapeDtypeStruct((NW, N), jnp.float32),
        ],
        mesh=mesh,
        compiler_params=pltpu.CompilerParams(needs_layout_passes=False),
        scratch_types=[
            pltpu.VMEM((NP,), jnp.float32),
            pltpu.VMEM((NP,), jnp.float32),
            pltpu.VMEM((N,), jnp.float32),
            pltpu.VMEM((SUP,), jnp.int32),
            pltpu.VMEM((SUP,), jnp.int32),
            pltpu.VMEM((SUP,), jnp.int32),
            pltpu.VMEM((SUP,), jnp.int32),
            pltpu.VMEM((K,), jnp.int32),
            pltpu.VMEM((K,), jnp.int32),
            pltpu.VMEM((K,), jnp.float32),
            pltpu.VMEM((K,), jnp.float32),
            pltpu.VMEM((K, D), jnp.float32),
            pltpu.VMEM((K, D), jnp.float32),
            pltpu.VMEM_SHARED((NP, D), jnp.float32),
            pltpu.SemaphoreType.DMA,
            pltpu.SemaphoreType.DMA,
            pltpu.SemaphoreType.DMA,
            pltpu.SemaphoreType.DMA,
        ],
    )
    return fn(src, dst, s_out, hmsg)


def _post_body(acc_ref, den_ref, feat_ref, w2_ref, w2b_ref, out_ref):
    acc = acc_ref[0] + acc_ref[1]
    den = jnp.sum(den_ref[...], axis=0)[:, None]
    h = acc / (den + 1e-9)
    f = feat_ref[...]
    w2p = lax.dot_general(f * h, w2_ref[...], (((1,), (1,)), ((), ())),
                          preferred_element_type=jnp.float32) + w2b_ref[...]
    o = f + h + w2p
    out_ref[...] = jnp.where(o >= 0.0, o, 0.2 * o)


def _tc_post(acc, den, featp, W2_w, W2_b, N):
    NP = featp.shape[0]
    grid = (NP // BS,)
    return pl.pallas_call(
        _post_body,
        grid=grid,
        in_specs=[
            pl.BlockSpec((NC, BS, D), lambda i: (0, i, 0)),
            pl.BlockSpec((NW, BS), lambda i: (0, i)),
            pl.BlockSpec((BS, D), lambda i: (i, 0)),
            pl.BlockSpec((D, D), lambda i: (0, 0)),
            pl.BlockSpec((1, D), lambda i: (0, 0)),
        ],
        out_specs=pl.BlockSpec((BS, D), lambda i: (i, 0)),
        out_shape=jax.ShapeDtypeStruct((N, D), jnp.float32),
    )(acc, den, featp, W2_w, W2_b)


def kernel(indices, features, num_nodes, W1_w, W1_b, W2_w, W2_b, Watt_w, Watt_b, a):
    N = features.shape[0]
    E = indices.shape[1]
    NP = -(-N // BS) * BS
    per_s = -(-E // NS)
    g = 2 * SUP
    EPW0 = max(g, int(round(0.5 * per_s / g)) * g)
    EPW1 = -(-(per_s - EPW0) // g) * g
    EP = NS * (EPW0 + EPW1)

    idxp = jnp.pad(indices.astype(jnp.int32), ((0, 0), (0, EP - E)))
    featp = jnp.pad(features.astype(jnp.float32), ((0, NP - N), (0, 0)))
    a1 = a[:D, 0].reshape(1, D).astype(jnp.float32)
    a2 = a[D:, 0].reshape(1, D).astype(jnp.float32)

    hmsg, s_out = _tc_pre(featp, Watt_w, Watt_b.reshape(1, D), a1, a2,
                          W1_w, W1_b.reshape(1, D))
    acc, den = _sc_edge_call(idxp[0], idxp[1], s_out, hmsg, NP, N, E, EPW0, EPW1)
    out = _tc_post(acc, den, featp, W2_w, W2_b.reshape(1, D), N)
    return out


# speedup vs baseline: 2.1687x; 1.1276x over previous
import functools

import jax
import jax.numpy as jnp
from jax import lax
from jax.experimental import pallas as pl
from jax.experimental.pallas import tpu as pltpu
from jax.experimental.pallas import tpu_sc as plsc

D = 128
BS = 512
K = 64
NC, NS = 2, 16
NW = NC * NS


def _pre_body(feat_ref, watt_ref, wattb_ref, a1_ref, a2_ref, w1_ref, w1b_ref,
              hmsg_ref, s_ref):
    f = feat_ref[...]
    ht = lax.dot_general(f, watt_ref[...], (((1,), (1,)), ((), ())),
                         preferred_element_type=jnp.float32) + wattb_ref[...]
    s1 = lax.dot_general(a1_ref[...], ht, (((1,), (1,)), ((), ())),
                         preferred_element_type=jnp.float32)
    s2 = lax.dot_general(a2_ref[...], ht, (((1,), (1,)), ((), ())),
                         preferred_element_type=jnp.float32)
    s_ref[0:1, :] = s1
    s_ref[1:2, :] = s2
    s_ref[2:8, :] = jnp.zeros((6, s1.shape[1]), jnp.float32)
    hmsg_ref[...] = lax.dot_general(f, w1_ref[...], (((1,), (1,)), ((), ())),
                                    preferred_element_type=jnp.float32) + w1b_ref[...]


def _tc_pre(featp, Watt_w, Watt_b, a1, a2, W1_w, W1_b):
    NP = featp.shape[0]
    grid = (NP // BS,)
    return pl.pallas_call(
        _pre_body,
        grid=grid,
        in_specs=[
            pl.BlockSpec((BS, D), lambda i: (i, 0)),
            pl.BlockSpec((D, D), lambda i: (0, 0)),
            pl.BlockSpec((1, D), lambda i: (0, 0)),
            pl.BlockSpec((1, D), lambda i: (0, 0)),
            pl.BlockSpec((1, D), lambda i: (0, 0)),
            pl.BlockSpec((D, D), lambda i: (0, 0)),
            pl.BlockSpec((1, D), lambda i: (0, 0)),
        ],
        out_specs=[
            pl.BlockSpec((BS, D), lambda i: (i, 0)),
            pl.BlockSpec((8, BS), lambda i: (0, i)),
        ],
        out_shape=[
            jax.ShapeDtypeStruct((NP, D), jnp.float32),
            jax.ShapeDtypeStruct((8, NP), jnp.float32),
        ],
    )(featp, Watt_w, Watt_b, a1, a2, W1_w, W1_b)


SUP = 128
NCK = SUP // K


def _sc_edge_call(src, dst, s_out, hmsg, NP, N, E, EPW0, EPW1):
    npairs0, npairs1 = EPW0 // (2 * SUP), EPW1 // (2 * SUP)
    stripe = NP // NS
    mesh = plsc.VectorSubcoreMesh(core_axis_name="c", subcore_axis_name="s")

    def body(src_hbm, dst_hbm, s_hbm, hmsg_hbm, acc_out, den_out,
             s1_v, s2_v, den_v, ssA, dsA, ssB, dsB, db0, db1, w0, w1,
             rows0, rows1, acc_sh, gsem0, gsem1, ldA, ldB):
        dbs, wvs, rows, gsems = [db0, db1], [w0, w1], [rows0, rows1], [gsem0, gsem1]
        c = lax.axis_index("c")
        s = lax.axis_index("s")
        wid = s * NC + c
        ebase = jnp.where(c == 0, s * EPW0, NS * EPW0 + s * EPW1)
        npairs_c = jnp.where(c == 0, npairs0, npairs1)

        pltpu.sync_copy(s_hbm.at[0], s1_v)
        pltpu.sync_copy(s_hbm.at[1], s2_v)

        def _zden(i, _):
            den_v[pl.ds(i * 16, 16)] = jnp.zeros((16,), jnp.float32)
            return _
        lax.fori_loop(0, N // 16, _zden, 0)

        def _zrow(i, _):
            for j in range(D // 16):
                rows0[i, pl.ds(j * 16, 16)] = jnp.zeros((16,), jnp.float32)
            return _
        lax.fori_loop(0, K, _zrow, 0)
        for t in range(stripe // K):
            pltpu.sync_copy(rows0, acc_sh.at[pl.ds(s * stripe + t * K, K)])
        plsc.subcore_barrier()

        iota16 = lax.broadcasted_iota(jnp.int32, (16,), 0)

        def scalar_phase(gbase, ss_c, ds_c, ck, par):
            for j in range(K // 16):
                off = ck * K + j * 16
                si = ss_c[pl.ds(off, 16)]
                di = ds_c[pl.ds(off, 16)]
                z = plsc.load_gather(s1_v, [si]) + plsc.load_gather(s2_v, [di])
                z = jnp.where(z >= 0.0, z, 0.2 * z)
                w = jnp.exp(z)
                gid = gbase + off + iota16
                w = jnp.where(gid < E, w, 0.0)
                wvs[par][pl.ds(j * 16, 16)] = w
                dbs[par][pl.ds(j * 16, 16)] = di
                plsc.addupdate_scatter(den_v, [di], w)

        def scale_rows(par):
            rws, wv = rows[par], wvs[par]

            def scale(k, _s):
                for u in range(4):
                    kk = k * 4 + u
                    wk = plsc.load_gather(wv, [lax.broadcast(kk, (16,))])
                    for j in range(D // 16):
                        rws[kk, pl.ds(j * 16, 16)] = rws[kk, pl.ds(j * 16, 16)] * wk
                return _s
            lax.fori_loop(0, K // 4, scale, 0)

        def do_super(sbase, ss_c, ds_c, ss_n, ds_n, ld_n, fire_guard):
            @pl.when(fire_guard)
            def _prefetch():
                nb = sbase + SUP
                pltpu.async_copy(src_hbm.at[pl.ds(nb, SUP)], ss_n, ld_n)
                pltpu.async_copy(dst_hbm.at[pl.ds(nb, SUP)], ds_n, ld_n)

            for ck in range(NCK):
                par = ck % 2
                if ck < NCK - 1:
                    pltpu.async_copy(
                        hmsg_hbm.at[ss_c.at[pl.ds((ck + 1) * K, K)]],
                        rows[1 - par], gsems[1 - par])
                else:
                    @pl.when(fire_guard)
                    def _boundary():
                        pltpu.make_async_copy(src_hbm.at[pl.ds(0, SUP)],
                                              ss_n, ld_n).wait()
                        pltpu.make_async_copy(dst_hbm.at[pl.ds(0, SUP)],
                                              ds_n, ld_n).wait()
                        pltpu.async_copy(hmsg_hbm.at[ss_n.at[pl.ds(0, K)]],
                                         rows[1 - par], gsems[1 - par])
                scalar_phase(sbase, ss_c, ds_c, ck, par)
                pltpu.make_async_copy(hmsg_hbm.at[ss_c.at[pl.ds(ck * K, K)]],
                                      rows[par], gsems[par]).wait()
                scale_rows(par)
                pltpu.sync_copy(rows[par], acc_sh.at[dbs[par]], add=True)

        pltpu.sync_copy(src_hbm.at[pl.ds(ebase, SUP)], ssA)
        pltpu.sync_copy(dst_hbm.at[pl.ds(ebase, SUP)], dsA)
        pltpu.async_copy(hmsg_hbm.at[ssA.at[pl.ds(0, K)]], rows0, gsem0)

        def pair_body(i, _):
            base_a = ebase + (2 * i) * SUP
            do_super(base_a, ssA, dsA, ssB, dsB, ldB, i >= 0)
            do_super(base_a + SUP, ssB, dsB, ssA, dsA, ldA, i + 1 < npairs_c)
            return _
        lax.fori_loop(0, npairs_c, pair_body, 0)

        plsc.subcore_barrier()
        pltpu.sync_copy(acc_sh.at[pl.ds(s * stripe, stripe)],
                        acc_out.at[c, pl.ds(s * stripe, stripe)])
        pltpu.sync_copy(den_v, den_out.at[wid])

    fn = pl.kernel(
        body,
        out_type=[
            jax.ShapeDtypeStruct((NC, NP, D), jnp.float32),
            jax.ShapeDtypeStruct((NW, N), jnp.float32),
        ],
        mesh=mesh,
        compiler_params=pltpu.CompilerParams(needs_layout_passes=False),
        scratch_types=[
            pltpu.VMEM((NP,), jnp.float32),
            pltpu.VMEM((NP,), jnp.float32),
            pltpu.VMEM((N,), jnp.float32),
            pltpu.VMEM((SUP,), jnp.int32),
            pltpu.VMEM((SUP,), jnp.int32),
            pltpu.VMEM((SUP,), jnp.int32),
            pltpu.VMEM((SUP,), jnp.int32),
            pltpu.VMEM((K,), jnp.int32),
            pltpu.VMEM((K,), jnp.int32),
            pltpu.VMEM((K,), jnp.float32),
            pltpu.VMEM((K,), jnp.float32),
            pltpu.VMEM((K, D), jnp.float32),
            pltpu.VMEM((K, D), jnp.float32),
            pltpu.VMEM_SHARED((NP, D), jnp.float32),
            pltpu.SemaphoreType.DMA,
            pltpu.SemaphoreType.DMA,
            pltpu.SemaphoreType.DMA,
            pltpu.SemaphoreType.DMA,
        ],
    )
    return fn(src, dst, s_out, hmsg)


def _post_body(acc_ref, den_ref, feat_ref, w2_ref, w2b_ref, out_ref):
    acc = acc_ref[0] + acc_ref[1]
    den = jnp.sum(den_ref[...], axis=0)[:, None]
    h = acc / (den + 1e-9)
    f = feat_ref[...]
    w2p = lax.dot_general(f * h, w2_ref[...], (((1,), (1,)), ((), ())),
                          preferred_element_type=jnp.float32) + w2b_ref[...]
    o = f + h + w2p
    out_ref[...] = jnp.where(o >= 0.0, o, 0.2 * o)


def _tc_post(acc, den, featp, W2_w, W2_b, N):
    NP = featp.shape[0]
    grid = (NP // BS,)
    return pl.pallas_call(
        _post_body,
        grid=grid,
        in_specs=[
            pl.BlockSpec((NC, BS, D), lambda i: (0, i, 0)),
            pl.BlockSpec((NW, BS), lambda i: (0, i)),
            pl.BlockSpec((BS, D), lambda i: (i, 0)),
            pl.BlockSpec((D, D), lambda i: (0, 0)),
            pl.BlockSpec((1, D), lambda i: (0, 0)),
        ],
        out_specs=pl.BlockSpec((BS, D), lambda i: (i, 0)),
        out_shape=jax.ShapeDtypeStruct((N, D), jnp.float32),
    )(acc, den, featp, W2_w, W2_b)


def kernel(indices, features, num_nodes, W1_w, W1_b, W2_w, W2_b, Watt_w, Watt_b, a):
    N = features.shape[0]
    E = indices.shape[1]
    NP = -(-N // BS) * BS
    per_s = -(-E // NS)
    g = 2 * SUP
    EPW0 = max(g, int(round(0.64 * per_s / g)) * g)
    EPW1 = -(-(per_s - EPW0) // g) * g
    EP = NS * (EPW0 + EPW1)

    idxp = jnp.pad(indices.astype(jnp.int32), ((0, 0), (0, EP - E)))
    featp = jnp.pad(features.astype(jnp.float32), ((0, NP - N), (0, 0)))
    a1 = a[:D, 0].reshape(1, D).astype(jnp.float32)
    a2 = a[D:, 0].reshape(1, D).astype(jnp.float32)

    hmsg, s_out = _tc_pre(featp, Watt_w, Watt_b.reshape(1, D), a1, a2,
                          W1_w, W1_b.reshape(1, D))
    acc, den = _sc_edge_call(idxp[0], idxp[1], s_out, hmsg, NP, N, E, EPW0, EPW1)
    out = _tc_post(acc, den, featp, W2_w, W2_b.reshape(1, D), N)
    return out


# speedup vs baseline: 2.3089x; 1.0646x over previous
import functools

import jax
import jax.numpy as jnp
from jax import lax
from jax.experimental import pallas as pl
from jax.experimental.pallas import tpu as pltpu
from jax.experimental.pallas import tpu_sc as plsc

D = 128
BS = 512
K = 64
NC, NS = 2, 16
NW = NC * NS


def _pre_body(feat_ref, watt_ref, wattb_ref, a1_ref, a2_ref, w1_ref, w1b_ref,
              hmsg_ref, s_ref):
    f = feat_ref[...]
    ht = lax.dot_general(f, watt_ref[...], (((1,), (1,)), ((), ())),
                         preferred_element_type=jnp.float32) + wattb_ref[...]
    s1 = lax.dot_general(a1_ref[...], ht, (((1,), (1,)), ((), ())),
                         preferred_element_type=jnp.float32)
    s2 = lax.dot_general(a2_ref[...], ht, (((1,), (1,)), ((), ())),
                         preferred_element_type=jnp.float32)
    s_ref[0:1, :] = s1
    s_ref[1:2, :] = s2
    s_ref[2:8, :] = jnp.zeros((6, s1.shape[1]), jnp.float32)
    hmsg_ref[...] = lax.dot_general(f, w1_ref[...], (((1,), (1,)), ((), ())),
                                    preferred_element_type=jnp.float32) + w1b_ref[...]


def _tc_pre(featp, Watt_w, Watt_b, a1, a2, W1_w, W1_b):
    NP = featp.shape[0]
    grid = (NP // BS,)
    return pl.pallas_call(
        _pre_body,
        grid=grid,
        in_specs=[
            pl.BlockSpec((BS, D), lambda i: (i, 0)),
            pl.BlockSpec((D, D), lambda i: (0, 0)),
            pl.BlockSpec((1, D), lambda i: (0, 0)),
            pl.BlockSpec((1, D), lambda i: (0, 0)),
            pl.BlockSpec((1, D), lambda i: (0, 0)),
            pl.BlockSpec((D, D), lambda i: (0, 0)),
            pl.BlockSpec((1, D), lambda i: (0, 0)),
        ],
        out_specs=[
            pl.BlockSpec((BS, D), lambda i: (i, 0)),
            pl.BlockSpec((8, BS), lambda i: (0, i)),
        ],
        out_shape=[
            jax.ShapeDtypeStruct((NP, D), jnp.float32),
            jax.ShapeDtypeStruct((8, NP), jnp.float32),
        ],
    )(featp, Watt_w, Watt_b, a1, a2, W1_w, W1_b)


SUP = 128
NCK = SUP // K


def _sc_edge_call(src, dst, s_out, hmsg, NP, N, E, EPW0, EPW1):
    npairs0, npairs1 = EPW0 // (2 * SUP), EPW1 // (2 * SUP)
    stripe = NP // NS
    mesh = plsc.VectorSubcoreMesh(core_axis_name="c", subcore_axis_name="s")

    def body(src_hbm, dst_hbm, s_hbm, hmsg_hbm, acc_out, den_out,
             s1_v, s2_v, den_v, ssA, dsA, ssB, dsB, db0, db1, w0, w1,
             rows0, rows1, acc_sh, gsem0, gsem1, ldA, ldB):
        dbs, wvs, rows, gsems = [db0, db1], [w0, w1], [rows0, rows1], [gsem0, gsem1]
        c = lax.axis_index("c")
        s = lax.axis_index("s")
        wid = s * NC + c
        ebase = jnp.where(c == 0, s * EPW0, NS * EPW0 + s * EPW1)
        npairs_c = jnp.where(c == 0, npairs0, npairs1)

        pltpu.sync_copy(s_hbm.at[0], s1_v)
        pltpu.sync_copy(s_hbm.at[1], s2_v)

        def _zden(i, _):
            den_v[pl.ds(i * 16, 16)] = jnp.zeros((16,), jnp.float32)
            return _
        lax.fori_loop(0, N // 16, _zden, 0)

        def _zrow(i, _):
            for j in range(D // 16):
                rows0[i, pl.ds(j * 16, 16)] = jnp.zeros((16,), jnp.float32)
            return _
        lax.fori_loop(0, K, _zrow, 0)
        for t in range(stripe // K):
            pltpu.sync_copy(rows0, acc_sh.at[pl.ds(s * stripe + t * K, K)])
        plsc.subcore_barrier()

        iota16 = lax.broadcasted_iota(jnp.int32, (16,), 0)

        def scalar_phase(gbase, ss_c, ds_c, ck, par):
            for j in range(K // 16):
                off = ck * K + j * 16
                si = ss_c[pl.ds(off, 16)]
                di = ds_c[pl.ds(off, 16)]
                z = plsc.load_gather(s1_v, [si]) + plsc.load_gather(s2_v, [di])
                z = jnp.where(z >= 0.0, z, 0.2 * z)
                w = jnp.exp(z)
                gid = gbase + off + iota16
                w = jnp.where(gid < E, w, 0.0)
                wvs[par][pl.ds(j * 16, 16)] = w
                dbs[par][pl.ds(j * 16, 16)] = di
                plsc.addupdate_scatter(den_v, [di], w)

        def scale_rows(par):
            rws, wv = rows[par], wvs[par]

            def scale(k, _s):
                for u in range(4):
                    kk = k * 4 + u
                    wk = plsc.load_gather(wv, [lax.broadcast(kk, (16,))])
                    for j in range(D // 16):
                        rws[kk, pl.ds(j * 16, 16)] = rws[kk, pl.ds(j * 16, 16)] * wk
                return _s
            lax.fori_loop(0, K // 4, scale, 0)

        def do_super(sbase, ss_c, ds_c, ss_n, ds_n, ld_n, fire_guard):
            @pl.when(fire_guard)
            def _prefetch():
                nb = sbase + SUP
                pltpu.async_copy(src_hbm.at[pl.ds(nb, SUP)], ss_n, ld_n)
                pltpu.async_copy(dst_hbm.at[pl.ds(nb, SUP)], ds_n, ld_n)

            for ck in range(NCK):
                par = ck % 2
                if ck < NCK - 1:
                    pltpu.async_copy(
                        hmsg_hbm.at[ss_c.at[pl.ds((ck + 1) * K, K)]],
                        rows[1 - par], gsems[1 - par])
                else:
                    @pl.when(fire_guard)
                    def _boundary():
                        pltpu.make_async_copy(src_hbm.at[pl.ds(0, SUP)],
                                              ss_n, ld_n).wait()
                        pltpu.make_async_copy(dst_hbm.at[pl.ds(0, SUP)],
                                              ds_n, ld_n).wait()
                        pltpu.async_copy(hmsg_hbm.at[ss_n.at[pl.ds(0, K)]],
                                         rows[1 - par], gsems[1 - par])
                scalar_phase(sbase, ss_c, ds_c, ck, par)
                pltpu.make_async_copy(hmsg_hbm.at[ss_c.at[pl.ds(ck * K, K)]],
                                      rows[par], gsems[par]).wait()
                scale_rows(par)
                pltpu.sync_copy(rows[par], acc_sh.at[dbs[par]], add=True)

        pltpu.sync_copy(src_hbm.at[pl.ds(ebase, SUP)], ssA)
        pltpu.sync_copy(dst_hbm.at[pl.ds(ebase, SUP)], dsA)
        pltpu.async_copy(hmsg_hbm.at[ssA.at[pl.ds(0, K)]], rows0, gsem0)

        def pair_body(i, _):
            base_a = ebase + (2 * i) * SUP
            do_super(base_a, ssA, dsA, ssB, dsB, ldB, i >= 0)
            do_super(base_a + SUP, ssB, dsB, ssA, dsA, ldA, i + 1 < npairs_c)
            return _
        lax.fori_loop(0, npairs_c, pair_body, 0)

        plsc.subcore_barrier()
        pltpu.sync_copy(acc_sh.at[pl.ds(s * stripe, stripe)],
                        acc_out.at[c, pl.ds(s * stripe, stripe)])
        pltpu.sync_copy(den_v, den_out.at[wid])

    fn = pl.kernel(
        body,
        out_type=[
            jax.ShapeDtypeStruct((NC, NP, D), jnp.float32),
            jax.ShapeDtypeStruct((NW, N), jnp.float32),
        ],
        mesh=mesh,
        compiler_params=pltpu.CompilerParams(needs_layout_passes=False),
        scratch_types=[
            pltpu.VMEM((NP,), jnp.float32),
            pltpu.VMEM((NP,), jnp.float32),
            pltpu.VMEM((N,), jnp.float32),
            pltpu.VMEM((SUP,), jnp.int32),
            pltpu.VMEM((SUP,), jnp.int32),
            pltpu.VMEM((SUP,), jnp.int32),
            pltpu.VMEM((SUP,), jnp.int32),
            pltpu.VMEM((K,), jnp.int32),
            pltpu.VMEM((K,), jnp.int32),
            pltpu.VMEM((K,), jnp.float32),
            pltpu.VMEM((K,), jnp.float32),
            pltpu.VMEM((K, D), jnp.float32),
            pltpu.VMEM((K, D), jnp.float32),
            pltpu.VMEM_SHARED((NP, D), jnp.float32),
            pltpu.SemaphoreType.DMA,
            pltpu.SemaphoreType.DMA,
            pltpu.SemaphoreType.DMA,
            pltpu.SemaphoreType.DMA,
        ],
    )
    return fn(src, dst, s_out, hmsg)


def _post_body(acc_ref, den_ref, feat_ref, w2_ref, w2b_ref, out_ref):
    acc = acc_ref[0] + acc_ref[1]
    den = jnp.sum(den_ref[...], axis=0)[:, None]
    h = acc / (den + 1e-9)
    f = feat_ref[...]
    w2p = lax.dot_general(f * h, w2_ref[...], (((1,), (1,)), ((), ())),
                          preferred_element_type=jnp.float32) + w2b_ref[...]
    o = f + h + w2p
    out_ref[...] = jnp.where(o >= 0.0, o, 0.2 * o)


def _tc_post(acc, den, featp, W2_w, W2_b, N):
    NP = featp.shape[0]
    grid = (NP // BS,)
    return pl.pallas_call(
        _post_body,
        grid=grid,
        in_specs=[
            pl.BlockSpec((NC, BS, D), lambda i: (0, i, 0)),
            pl.BlockSpec((NW, BS), lambda i: (0, i)),
            pl.BlockSpec((BS, D), lambda i: (i, 0)),
            pl.BlockSpec((D, D), lambda i: (0, 0)),
            pl.BlockSpec((1, D), lambda i: (0, 0)),
        ],
        out_specs=pl.BlockSpec((BS, D), lambda i: (i, 0)),
        out_shape=jax.ShapeDtypeStruct((N, D), jnp.float32),
    )(acc, den, featp, W2_w, W2_b)


def kernel(indices, features, num_nodes, W1_w, W1_b, W2_w, W2_b, Watt_w, Watt_b, a):
    N = features.shape[0]
    E = indices.shape[1]
    NP = -(-N // BS) * BS
    per_s = -(-E // NS)
    g = 2 * SUP
    EPW0 = max(g, int(round(0.70 * per_s / g)) * g)
    EPW1 = -(-(per_s - EPW0) // g) * g
    EP = NS * (EPW0 + EPW1)

    idxp = jnp.pad(indices.astype(jnp.int32), ((0, 0), (0, EP - E)))
    featp = jnp.pad(features.astype(jnp.float32), ((0, NP - N), (0, 0)))
    a1 = a[:D, 0].reshape(1, D).astype(jnp.float32)
    a2 = a[D:, 0].reshape(1, D).astype(jnp.float32)

    hmsg, s_out = _tc_pre(featp, Watt_w, Watt_b.reshape(1, D), a1, a2,
                          W1_w, W1_b.reshape(1, D))
    acc, den = _sc_edge_call(idxp[0], idxp[1], s_out, hmsg, NP, N, E, EPW0, EPW1)
    out = _tc_post(acc, den, featp, W2_w, W2_b.reshape(1, D), N)
    return out
